# baseline (speedup 1.0000x reference)
import jax
import jax.numpy as jnp
from jax import lax
from jax.experimental import pallas as pl
from jax.experimental.pallas import tpu as pltpu
from jax.experimental.pallas import tpu_sc as plsc

RESO = 128
CAP = RESO ** 3
D_OUT = 28
D_PAD = 32
NC, NS = 2, 16
NW = NC * NS
K = 128
N_PTS = 1000000
N_CHUNKS = -(-N_PTS // K)
MAIN = (N_CHUNKS // NW) & ~1
EXTRA = N_CHUNKS - MAIN * NW
REM = N_PTS - (N_CHUNKS - 1) * K


def _sc_body(pts, tbl, out, cb0, cb1, idxs0, idxs1, wts0, wts1,
             rows0, rows1, outb0, outb1, sem_c, sem_g0, sem_g1,
             sem_o0, sem_o1):
    cid = lax.axis_index("c")
    sid = lax.axis_index("s")
    wid = sid * NC + cid

    def base_of(t):
        return (wid + t * NW) * K

    def fire_coords(t, cb):
        pltpu.async_copy(pts.at[pl.ds(base_of(t), K)], cb, sem_c)

    def wait_coords(cb):
        pltpu.make_async_copy(pts.at[pl.ds(0, K)], cb, sem_c).wait()

    def prep(cb, idxs, wts, rows, sem_g):
        for j in range(K // 16):
            sl = pl.ds(j * 16, 16)
            pid = j * 16 + lax.iota(jnp.int32, 16)

            def axis_calc(dim):
                v = plsc.load_gather(cb, [pid, jnp.full((16,), dim, jnp.int32)])
                t = 63.5 + 64.0 * v
                t = jnp.minimum(jnp.maximum(t, 0.0), float(RESO - 1))
                l = t.astype(jnp.int32)
                l = jnp.minimum(l, RESO - 2)
                wb = t - l.astype(jnp.float32)
                return l, wb

            lx, wbx = axis_calc(0)
            ly, wby = axis_calc(1)
            lz, wbz = axis_calc(2)
            ib = lx * (RESO * RESO) + ly * RESO + lz
            wx = (1.0 - wbx, wbx)
            wy = (1.0 - wby, wby)
            wz = (1.0 - wbz, wbz)
            for c8 in range(8):
                dx, dy, dz = (c8 >> 2) & 1, (c8 >> 1) & 1, c8 & 1
                idxs[c8, sl] = ib + (dx * RESO * RESO + dy * RESO + dz)
                wts[c8, sl] = wx[dx] * wy[dy] * wz[dz]
        for c8 in range(8):
            pltpu.async_copy(tbl.at[idxs.at[c8]], rows.at[pl.ds(c8 * K, K)],
                             sem_g)

    def wait_gathers(rows, sem_g):
        pltpu.make_async_copy(tbl.at[pl.ds(0, 8 * K)], rows, sem_g).wait()

    def blend_compute(rows, wts, outb):
        def blend_j(j, carry):
            iot = lax.iota(jnp.int32, 16)
            rowid = j * 16 + iot
            rids = [rowid + c8 * K for c8 in range(8)]
            wv = [wts[c8, pl.ds(j * 16, 16)] for c8 in range(8)]
            for r in range(D_OUT):
                bc = iot + r
                chv = jnp.where(bc >= D_OUT, bc - D_OUT, bc)
                acc = plsc.load_gather(rows, [rids[0], chv]) * wv[0]
                for c8 in range(1, 8):
                    acc = acc + plsc.load_gather(rows, [rids[c8], chv]) * wv[c8]
                plsc.store_scatter(outb, [rowid, chv], acc)
            return carry

        lax.fori_loop(0, K // 16, blend_j, 0)

    def blend(t, rows, wts, outb, sem_o, wait_store):
        @pl.when(wait_store)
        def _():
            pltpu.make_async_copy(outb, out.at[pl.ds(0, K)], sem_o).wait()

        blend_compute(rows, wts, outb)
        pltpu.async_copy(outb, out.at[pl.ds(base_of(t), K)], sem_o)

    fire_coords(0, cb0)
    wait_coords(cb0)
    prep(cb0, idxs0, wts0, rows0, sem_g0)
    fire_coords(1, cb1)

    def pair(i, carry):
        gg = i * 2
        more = gg < MAIN - 2
        wait_coords(cb1)
        prep(cb1, idxs1, wts1, rows1, sem_g1)

        @pl.when(more)
        def _():
            fire_coords(gg + 2, cb0)

        wait_gathers(rows0, sem_g0)
        blend(gg, rows0, wts0, outb0, sem_o0, gg >= 2)

        @pl.when(more)
        def _():
            wait_coords(cb0)
            prep(cb0, idxs0, wts0, rows0, sem_g0)
            fire_coords(gg + 3, cb1)

        wait_gathers(rows1, sem_g1)
        blend(gg + 1, rows1, wts1, outb1, sem_o1, gg >= 2)
        return carry

    lax.fori_loop(0, MAIN // 2, pair, 0)

    @pl.when(wid < EXTRA)
    def _():
        ebase = (MAIN * NW + wid) * K
        last = wid == EXTRA - 1

        @pl.when(last)
        def _():
            pltpu.async_copy(pts.at[pl.ds(ebase, REM)], cb0.at[pl.ds(0, REM)],
                             sem_c)
            pltpu.make_async_copy(pts.at[pl.ds(0, REM)],
                                  cb0.at[pl.ds(0, REM)], sem_c).wait()

        @pl.when(jnp.logical_not(last))
        def _():
            pltpu.async_copy(pts.at[pl.ds(ebase, K)], cb0, sem_c)
            wait_coords(cb0)

        prep(cb0, idxs0, wts0, rows0, sem_g0)
        wait_gathers(rows0, sem_g0)
        pltpu.make_async_copy(outb0, out.at[pl.ds(0, K)], sem_o0).wait()
        blend_compute(rows0, wts0, outb0)

        @pl.when(last)
        def _():
            pltpu.async_copy(outb0.at[pl.ds(0, REM)],
                             out.at[pl.ds(ebase, REM)], sem_o0)
            pltpu.make_async_copy(outb0.at[pl.ds(0, REM)],
                                  out.at[pl.ds(0, REM)], sem_o0).wait()

        @pl.when(jnp.logical_not(last))
        def _():
            pltpu.async_copy(outb0, out.at[pl.ds(ebase, K)], sem_o0)
            pltpu.make_async_copy(outb0, out.at[pl.ds(0, K)], sem_o0).wait()

    @pl.when(wid >= EXTRA)
    def _():
        pltpu.make_async_copy(outb0, out.at[pl.ds(0, K)], sem_o0).wait()

    pltpu.make_async_copy(outb1, out.at[pl.ds(0, K)], sem_o1).wait()


def kernel(points, density_data, sh_data):
    pad_cols = D_PAD - 1 - sh_data.shape[1]
    tbl = jnp.concatenate(
        [density_data, sh_data, jnp.zeros((CAP, pad_cols), jnp.float32)], axis=1
    )
    mesh = plsc.VectorSubcoreMesh(
        core_axis_name="c", subcore_axis_name="s", num_cores=NC, num_subcores=NS
    )
    out = pl.kernel(
        _sc_body,
        out_type=jax.ShapeDtypeStruct((N_PTS, D_OUT), jnp.float32),
        mesh=mesh,
        compiler_params=pltpu.CompilerParams(
            needs_layout_passes=False, use_tc_tiling_on_sc=False
        ),
        scratch_types=[
            pltpu.VMEM((K, 3), jnp.float32),
            pltpu.VMEM((K, 3), jnp.float32),
            pltpu.VMEM((8, K), jnp.int32),
            pltpu.VMEM((8, K), jnp.int32),
            pltpu.VMEM((8, K), jnp.float32),
            pltpu.VMEM((8, K), jnp.float32),
            pltpu.VMEM((8 * K, D_PAD), jnp.float32),
            pltpu.VMEM((8 * K, D_PAD), jnp.float32),
            pltpu.VMEM((K, D_OUT), jnp.float32),
            pltpu.VMEM((K, D_OUT), jnp.float32),
            pltpu.SemaphoreType.DMA,
            pltpu.SemaphoreType.DMA,
            pltpu.SemaphoreType.DMA,
            pltpu.SemaphoreType.DMA,
            pltpu.SemaphoreType.DMA,
        ],
    )(points, tbl)
    return out


# speedup vs baseline: 2.1535x; 2.1535x over previous
import jax
import jax.numpy as jnp
from jax import lax
from jax.experimental import pallas as pl
from jax.experimental.pallas import tpu as pltpu
from jax.experimental.pallas import tpu_sc as plsc

RESO = 128
CAP = RESO ** 3
D_OUT = 28
D_PAD = 32
NC, NS = 2, 16
NW = NC * NS
K = 128
ITERS = 246
B_PAD = ITERS * NW * K
B_W = B_PAD // NW


def _sc_body(coords, tbl, out, cb0, cb1, idxs0, idxs1, wts0, wts1,
             rows0, rows1, outb0, outb1, sem_c, sem_g0, sem_g1,
             sem_o0, sem_o1):
    cid = lax.axis_index("c")
    sid = lax.axis_index("s")
    wbase = (sid * NC + cid) * B_W

    def fire_coords(gq, cb):
        base = wbase + gq * K
        for i in range(3):
            pltpu.async_copy(coords.at[i, pl.ds(base, K)], cb.at[i], sem_c)

    def wait_coords(cb):
        pltpu.make_async_copy(coords.at[:, pl.ds(0, K)], cb, sem_c).wait()

    def prep(cb, idxs, wts, rows, sem_g):
        for j in range(K // 16):
            sl = pl.ds(j * 16, 16)

            def axis_calc(v):
                t = 63.5 + 64.0 * v
                t = jnp.minimum(jnp.maximum(t, 0.0), float(RESO - 1))
                l = t.astype(jnp.int32)
                l = jnp.minimum(l, RESO - 2)
                wb = t - l.astype(jnp.float32)
                return l, wb

            lx, wbx = axis_calc(cb[0, sl])
            ly, wby = axis_calc(cb[1, sl])
            lz, wbz = axis_calc(cb[2, sl])
            ib = lx * (RESO * RESO) + ly * RESO + lz
            wx = (1.0 - wbx, wbx)
            wy = (1.0 - wby, wby)
            wz = (1.0 - wbz, wbz)
            for c8 in range(8):
                dx, dy, dz = (c8 >> 2) & 1, (c8 >> 1) & 1, c8 & 1
                idxs[c8, sl] = ib + (dx * RESO * RESO + dy * RESO + dz)
                wts[c8, sl] = wx[dx] * wy[dy] * wz[dz]
        for c8 in range(8):
            pltpu.async_copy(tbl.at[idxs.at[c8]], rows.at[pl.ds(c8 * K, K)],
                             sem_g)

    def wait_gathers(rows, sem_g):
        pltpu.make_async_copy(tbl.at[pl.ds(0, 8 * K)], rows, sem_g).wait()

    def blend(gq, rows, wts, outb, sem_o, wait_store):
        @pl.when(wait_store)
        def _():
            pltpu.make_async_copy(outb, out.at[pl.ds(0, K)], sem_o).wait()

        def blend_j(j, carry):
            iot = lax.iota(jnp.int32, 16)
            rowid = j * 16 + iot
            rids = [rowid + c8 * K for c8 in range(8)]
            wv = [wts[c8, pl.ds(j * 16, 16)] for c8 in range(8)]
            for r in range(D_OUT):
                bc = iot + r
                chv = jnp.where(bc >= D_OUT, bc - D_OUT, bc)
                acc = plsc.load_gather(rows, [rids[0], chv]) * wv[0]
                for c8 in range(1, 8):
                    acc = acc + plsc.load_gather(rows, [rids[c8], chv]) * wv[c8]
                plsc.store_scatter(outb, [rowid, chv], acc)
            return carry

        lax.fori_loop(0, K // 16, blend_j, 0)
        pltpu.async_copy(outb, out.at[pl.ds(wbase + gq * K, K)], sem_o)

    fire_coords(0, cb0)
    wait_coords(cb0)
    prep(cb0, idxs0, wts0, rows0, sem_g0)
    fire_coords(1, cb1)

    def pair(i, carry):
        gg = i * 2
        more = gg < ITERS - 2
        wait_coords(cb1)
        prep(cb1, idxs1, wts1, rows1, sem_g1)

        @pl.when(more)
        def _():
            fire_coords(gg + 2, cb0)

        wait_gathers(rows0, sem_g0)
        blend(gg, rows0, wts0, outb0, sem_o0, gg >= 2)

        @pl.when(more)
        def _():
            wait_coords(cb0)
            prep(cb0, idxs0, wts0, rows0, sem_g0)
            fire_coords(gg + 3, cb1)

        wait_gathers(rows1, sem_g1)
        blend(gg + 1, rows1, wts1, outb1, sem_o1, gg >= 2)
        return carry

    lax.fori_loop(0, ITERS // 2, pair, 0)
    pltpu.make_async_copy(outb0, out.at[pl.ds(0, K)], sem_o0).wait()
    pltpu.make_async_copy(outb1, out.at[pl.ds(0, K)], sem_o1).wait()


def kernel(points, density_data, sh_data):
    n = points.shape[0]
    coords = jnp.pad(points, ((0, B_PAD - n), (0, 0))).T
    pad_cols = D_PAD - 1 - sh_data.shape[1]
    tbl = jnp.concatenate(
        [density_data, sh_data, jnp.zeros((CAP, pad_cols), jnp.float32)], axis=1
    )
    mesh = plsc.VectorSubcoreMesh(
        core_axis_name="c", subcore_axis_name="s", num_cores=NC, num_subcores=NS
    )
    out = pl.kernel(
        _sc_body,
        out_type=jax.ShapeDtypeStruct((B_PAD, D_OUT), jnp.float32),
        mesh=mesh,
        compiler_params=pltpu.CompilerParams(
            needs_layout_passes=False, use_tc_tiling_on_sc=False
        ),
        scratch_types=[
            pltpu.VMEM((3, K), jnp.float32),
            pltpu.VMEM((3, K), jnp.float32),
            pltpu.VMEM((8, K), jnp.int32),
            pltpu.VMEM((8, K), jnp.int32),
            pltpu.VMEM((8, K), jnp.float32),
            pltpu.VMEM((8, K), jnp.float32),
            pltpu.VMEM((8 * K, D_PAD), jnp.float32),
            pltpu.VMEM((8 * K, D_PAD), jnp.float32),
            pltpu.VMEM((K, D_OUT), jnp.float32),
            pltpu.VMEM((K, D_OUT), jnp.float32),
            pltpu.SemaphoreType.DMA,
            pltpu.SemaphoreType.DMA,
            pltpu.SemaphoreType.DMA,
            pltpu.SemaphoreType.DMA,
            pltpu.SemaphoreType.DMA,
        ],
    )(coords, tbl)
    return out[:n]
